# baseline (speedup 1.0000x reference)
import functools

import jax
import jax.numpy as jnp
from jax import lax
from jax.experimental import pallas as pl
from jax.experimental.pallas import tpu as pltpu
from jax.experimental.pallas import tpu_sc as plsc

_B = 4096
_L = 50
_D = 768
_N = _B * _L
_NC = 2
_NS = 16
_NW = _NC * _NS
_BPW = _N // _NW
_CHUNK = 64
_NCHUNK = _BPW // _CHUNK


def _sc_gather(idx3d, emo_emb):
    mesh = plsc.VectorSubcoreMesh(core_axis_name="c", subcore_axis_name="s")

    @functools.partial(
        pl.kernel,
        mesh=mesh,
        out_type=jax.ShapeDtypeStruct((_N, _D), jnp.float32),
        scratch_types=[
            pltpu.VMEM((_NCHUNK, _CHUNK), jnp.int32),
            pltpu.VMEM((_CHUNK, _D), jnp.float32),
            pltpu.SemaphoreType.DMA,
        ],
    )
    def k(table_hbm, idx_hbm, out_hbm, idx_v, rows_v, sem):
        wid = lax.axis_index("s") * _NC + lax.axis_index("c")
        base = wid * _BPW
        pltpu.sync_copy(idx_hbm.at[wid], idx_v)

        def body(c, carry):
            pltpu.async_copy(table_hbm.at[idx_v.at[c]], rows_v, sem).wait()
            pltpu.sync_copy(
                rows_v, out_hbm.at[pl.ds(base + c * _CHUNK, _CHUNK)])
            return carry

        lax.fori_loop(0, _NCHUNK, body, 0)

    return k(emo_emb, idx3d)


def kernel(detect_emo, emo_emb):
    idx = detect_emo.reshape(_N).astype(jnp.int32).reshape(_NW, _NCHUNK, _CHUNK)
    out = _sc_gather(idx, emo_emb)
    return out.reshape(_B, _L, _D)
